# Initial kernel scaffold; baseline (speedup 1.0000x reference)
import jax
import jax.numpy as jnp
from jax.experimental import pallas as pl


def kernel(y_pred, y_dash, x_f_train, invp_index, p_index):
    raise NotImplementedError("write your pallas kernel here")


# speedup vs baseline: 2.6228x; 2.6228x over previous
import functools

import jax
import jax.numpy as jnp
from jax import lax
from jax.experimental import pallas as pl
from jax.experimental.pallas import tpu as pltpu
from jax.experimental.pallas import tpu_sc as plsc

N_F = 64
N = N_F * N_F
NR = 9
BATCH = 128
NC, NS, L = 2, 16, 16
CORE = N // NS
EXT = 400
WIN = 544
B_PER_SC = BATCH // NC
NU = 0.08
ALPHA = 0.5


def _sc_loss_parts(y2, yd2, xt, invt, pt):
    mesh = plsc.VectorSubcoreMesh(core_axis_name="c", subcore_axis_name="s")

    @functools.partial(
        pl.kernel,
        out_type=jax.ShapeDtypeStruct((NC * NS * L,), jnp.float32),
        mesh=mesh,
        scratch_types=[
            pltpu.VMEM((WIN,), jnp.float32),
            pltpu.VMEM((CORE,), jnp.float32),
            pltpu.VMEM((NR * EXT,), jnp.int32),
            pltpu.VMEM((NR * CORE,), jnp.int32),
            pltpu.VMEM((NR * EXT,), jnp.float32),
            pltpu.VMEM((NR * EXT,), jnp.float32),
            pltpu.VMEM((EXT,), jnp.float32),
            pltpu.VMEM((EXT,), jnp.float32),
            pltpu.VMEM((WIN,), jnp.float32),
            pltpu.VMEM((WIN,), jnp.float32),
            pltpu.VMEM((4 * EXT,), jnp.float32),
            pltpu.VMEM((EXT,), jnp.float32),
            pltpu.VMEM((EXT,), jnp.float32),
            pltpu.VMEM((CORE,), jnp.float32),
            pltpu.VMEM((L,), jnp.float32),
        ],
        compiler_params=pltpu.CompilerParams(use_tc_tiling_on_sc=False,
                                             needs_layout_passes=False),
    )
    def k(y2h, yd2h, xth, invth, pth, outh,
          uw, udw, ie, ic, wxe, wye, Wxe, Wye, xw0, xw1, invr, uxe, uye,
          multv, pout):
        sc = lax.axis_index("c")
        tid = lax.axis_index("s")
        lo = pl.multiple_of(tid * CORE, 8)
        elo = pl.multiple_of(jnp.clip(lo - 72, 0, N - EXT), 8)
        s2 = pl.multiple_of(jnp.clip(elo - 72, 0, N - WIN), 8)
        off1 = elo - s2
        off2 = lo - elo
        off3 = lo - s2

        pltpu.sync_copy(xth.at[pl.ds(pl.multiple_of(s2, 8), WIN)], xw0)
        pltpu.sync_copy(xth.at[pl.ds(pl.multiple_of(N + s2, 8), WIN)], xw1)
        for kk in range(4):
            pltpu.sync_copy(invth.at[pl.ds(pl.multiple_of(kk * N + elo, 8), EXT)], invr.at[pl.ds(kk * EXT, EXT)])
        for p in range(NR):
            pltpu.sync_copy(pth.at[pl.ds(pl.multiple_of(p * N + elo, 8), EXT)], ie.at[pl.ds(p * EXT, EXT)])
            pltpu.sync_copy(pth.at[pl.ds(pl.multiple_of(p * N + lo, 8), CORE)], ic.at[pl.ds(p * CORE, CORE)])

        lanes = lax.iota(jnp.int32, L)
        for e in range(EXT // L):
            sl = pl.ds(e * L, L)
            xn0 = xw0[pl.ds(off1 + e * L, L)]
            xn1 = xw1[pl.ds(off1 + e * L, L)]
            ax = jnp.zeros((L,), jnp.float32)
            ay = jnp.zeros((L,), jnp.float32)
            for p in range(NR):
                psl = pl.ds(p * EXT + e * L, L)
                li = ie[psl] - s2
                ie[psl] = li
                xd0 = plsc.load_gather(xw0, [li]) - xn0
                xd1 = plsc.load_gather(xw1, [li]) - xn1
                wx = invr[pl.ds(0 * EXT + e * L, L)] * xd0 + invr[pl.ds(2 * EXT + e * L, L)] * xd1
                wy = invr[pl.ds(1 * EXT + e * L, L)] * xd0 + invr[pl.ds(3 * EXT + e * L, L)] * xd1
                wxe[psl] = wx
                wye[psl] = wy
                ax = ax + wx
                ay = ay + wy
            Wxe[sl] = ax
            Wye[sl] = ay
        for c in range(CORE // L):
            sl = pl.ds(c * L, L)
            for p in range(NR):
                csl = pl.ds(p * CORE + c * L, L)
                ic[csl] = ic[csl] - elo
            n = lo + c * L + lanes
            i = n // N_F
            j = n % N_F
            m = (jnp.where(i == 0, 1.0, 0.0)
                 + jnp.where(j == 0, 1.0, 0.0)
                 + jnp.where(j == N_F - 1, 1.0, 0.0))
            multv[sl] = m.astype(jnp.float32)

        sf = jnp.float32(1.0 / (BATCH * N))
        sb = jnp.float32(1.0 / (BATCH * 3 * N_F))

        def body(b, acc):
            bg = sc * B_PER_SC + b
            pltpu.sync_copy(y2h.at[pl.ds(pl.multiple_of(bg * N + s2, 8), WIN)], uw)
            pltpu.sync_copy(yd2h.at[pl.ds(pl.multiple_of(bg * N + lo, 8), CORE)], udw)
            for e in range(EXT // L):
                sl = pl.ds(e * L, L)
                ax = jnp.zeros((L,), jnp.float32)
                ay = jnp.zeros((L,), jnp.float32)
                for p in range(NR):
                    psl = pl.ds(p * EXT + e * L, L)
                    g = plsc.load_gather(uw, [ie[psl]])
                    ax = ax + g * wxe[psl]
                    ay = ay + g * wye[psl]
                un = uw[pl.ds(off1 + e * L, L)]
                uxe[sl] = ax - un * Wxe[sl]
                uye[sl] = ay - un * Wye[sl]
            for c in range(CORE // L):
                sl = pl.ds(c * L, L)
                esl = pl.ds(off2 + c * L, L)
                a2 = jnp.zeros((L,), jnp.float32)
                for p in range(NR):
                    g = plsc.load_gather(uxe, [ic[pl.ds(p * CORE + c * L, L)]])
                    a2 = a2 + g * wxe[pl.ds(p * EXT + off2 + c * L, L)]
                uxn = uxe[esl]
                uxx = a2 - uxn * Wxe[esl]
                un = uw[pl.ds(off3 + c * L, L)]
                fv = uye[esl] - NU * uxx - un * (1.0 - un) * (un + ALPHA)
                d = un - udw[sl]
                acc = acc + fv * fv * sf + multv[sl] * d * d * sb
            return acc

        acc = lax.fori_loop(0, B_PER_SC, body,
                            jnp.zeros((L,), jnp.float32))
        pout[...] = acc
        pltpu.sync_copy(pout, outh.at[pl.ds(pl.multiple_of((sc * NS + tid) * L, 8), L)])

    return k(y2, yd2, xt, invt, pt)


def _reduce_parts(parts):
    def red(x_ref, o_ref):
        o_ref[...] = jnp.sum(x_ref[...]).reshape(1, 1)

    out = pl.pallas_call(
        red, out_shape=jax.ShapeDtypeStruct((1, 1), jnp.float32),
    )(parts)
    return out[0, 0]


@jax.jit
def kernel(y_pred, y_dash, x_f_train, invp_index, p_index):
    y2 = y_pred.reshape(BATCH * N)
    yd2 = y_dash.reshape(BATCH * N)
    xt = x_f_train.T.reshape(2 * N)
    invt = invp_index.reshape(N, 4).T.reshape(4 * N)
    pt = p_index.astype(jnp.int32).T.reshape(NR * N)
    parts = _sc_loss_parts(y2, yd2, xt, invt, pt)
    return _reduce_parts(parts.reshape(NC * NS, L))


# speedup vs baseline: 6.7148x; 2.5601x over previous
import functools

import jax
import jax.numpy as jnp
from jax import lax
from jax.experimental import pallas as pl
from jax.experimental.pallas import tpu as pltpu
from jax.experimental.pallas import tpu_sc as plsc

N_F = 64
N = N_F * N_F
NR = 9
BATCH = 128
NC, NS, L = 2, 16, 16
CORE = N // NS
EXT = 400
WIN = 544
B_PER_SC = BATCH // NC
NU = 0.08
ALPHA = 0.5


def _sc_loss_parts(y2, yd2, xt, invt, pt):
    mesh = plsc.VectorSubcoreMesh(core_axis_name="c", subcore_axis_name="s")

    @functools.partial(
        pl.kernel,
        out_type=jax.ShapeDtypeStruct((NC * NS * L,), jnp.float32),
        mesh=mesh,
        scratch_types=[
            pltpu.VMEM((2 * WIN,), jnp.float32),
            pltpu.VMEM((2 * CORE,), jnp.float32),
            pltpu.VMEM((2 * NR * EXT,), jnp.int32),
            pltpu.VMEM((NR * CORE,), jnp.int32),
            pltpu.VMEM((NR * EXT,), jnp.float32),
            pltpu.VMEM((NR * EXT,), jnp.float32),
            pltpu.VMEM((EXT,), jnp.float32),
            pltpu.VMEM((EXT,), jnp.float32),
            pltpu.VMEM((WIN,), jnp.float32),
            pltpu.VMEM((WIN,), jnp.float32),
            pltpu.VMEM((4 * EXT,), jnp.float32),
            pltpu.VMEM((EXT,), jnp.float32),
            pltpu.VMEM((EXT,), jnp.float32),
            pltpu.VMEM((CORE,), jnp.float32),
            pltpu.VMEM((L,), jnp.float32),
            pltpu.SemaphoreType.DMA,
            pltpu.SemaphoreType.DMA,
            pltpu.SemaphoreType.DMA,
            pltpu.SemaphoreType.DMA,
        ],
        compiler_params=pltpu.CompilerParams(use_tc_tiling_on_sc=False,
                                             needs_layout_passes=False),
    )
    def k(y2h, yd2h, xth, invth, pth, outh,
          uw2, udw2, ie2, ic, wxe, wye, Wxe, Wye, xw0, xw1, invr, uxe, uye,
          multv, pout, semu0, semu1, semd0, semd1):
        sc = lax.axis_index("c")
        tid = lax.axis_index("s")
        lo = pl.multiple_of(tid * CORE, 8)
        elo = pl.multiple_of(jnp.clip(lo - 72, 0, N - EXT), 8)
        s2 = pl.multiple_of(jnp.clip(elo - 72, 0, N - WIN), 8)
        off1 = elo - s2
        off2 = lo - elo
        off3 = lo - s2
        semu = (semu0, semu1)
        semd = (semd0, semd1)

        pltpu.sync_copy(xth.at[pl.ds(pl.multiple_of(s2, 8), WIN)], xw0)
        pltpu.sync_copy(xth.at[pl.ds(pl.multiple_of(N + s2, 8), WIN)], xw1)
        for kk in range(4):
            pltpu.sync_copy(invth.at[pl.ds(pl.multiple_of(kk * N + elo, 8), EXT)],
                            invr.at[pl.ds(kk * EXT, EXT)])
        for p in range(NR):
            pltpu.sync_copy(pth.at[pl.ds(pl.multiple_of(p * N + elo, 8), EXT)],
                            ie2.at[pl.ds(p * EXT, EXT)])
            pltpu.sync_copy(pth.at[pl.ds(pl.multiple_of(p * N + lo, 8), CORE)],
                            ic.at[pl.ds(p * CORE, CORE)])

        lanes = lax.iota(jnp.int32, L)
        for e in range(EXT // L):
            sl = pl.ds(e * L, L)
            xn0 = xw0[pl.ds(off1 + e * L, L)]
            xn1 = xw1[pl.ds(off1 + e * L, L)]
            ax = jnp.zeros((L,), jnp.float32)
            ay = jnp.zeros((L,), jnp.float32)
            for p in range(NR):
                psl = pl.ds(p * EXT + e * L, L)
                li = ie2[psl] - s2
                ie2[psl] = li
                ie2[pl.ds(NR * EXT + p * EXT + e * L, L)] = li + WIN
                xd0 = plsc.load_gather(xw0, [li]) - xn0
                xd1 = plsc.load_gather(xw1, [li]) - xn1
                wx = invr[pl.ds(0 * EXT + e * L, L)] * xd0 + invr[pl.ds(2 * EXT + e * L, L)] * xd1
                wy = invr[pl.ds(1 * EXT + e * L, L)] * xd0 + invr[pl.ds(3 * EXT + e * L, L)] * xd1
                wxe[psl] = wx
                wye[psl] = wy
                ax = ax + wx
                ay = ay + wy
            Wxe[sl] = ax
            Wye[sl] = ay
        for c in range(CORE // L):
            sl = pl.ds(c * L, L)
            for p in range(NR):
                csl = pl.ds(p * CORE + c * L, L)
                ic[csl] = ic[csl] - elo
            n = lo + c * L + lanes
            i = n // N_F
            j = n % N_F
            m = (jnp.where(i == 0, 1.0, 0.0)
                 + jnp.where(j == 0, 1.0, 0.0)
                 + jnp.where(j == N_F - 1, 1.0, 0.0))
            multv[sl] = m.astype(jnp.float32)

        sf = jnp.float32(1.0 / (BATCH * N))
        sb = jnp.float32(1.0 / (BATCH * 3 * N_F))

        def u_src(b):
            bg = sc * B_PER_SC + b
            return y2h.at[pl.ds(pl.multiple_of(bg * N + s2, 8), WIN)]

        def d_src(b):
            bg = sc * B_PER_SC + b
            return yd2h.at[pl.ds(pl.multiple_of(bg * N + lo, 8), CORE)]

        def issue(b, slot):
            pltpu.async_copy(u_src(b), uw2.at[pl.ds(slot * WIN, WIN)], semu[slot])
            pltpu.async_copy(d_src(b), udw2.at[pl.ds(slot * CORE, CORE)], semd[slot])

        def drain(b, slot):
            pltpu.make_async_copy(u_src(b), uw2.at[pl.ds(slot * WIN, WIN)],
                                  semu[slot]).wait()
            pltpu.make_async_copy(d_src(b), udw2.at[pl.ds(slot * CORE, CORE)],
                                  semd[slot]).wait()

        def compute(slot, acc):
            ub = slot * WIN
            ib = slot * NR * EXT
            db = slot * CORE

            def p1(e, carry):
                sl = pl.ds(e * L, L)
                ax = jnp.zeros((L,), jnp.float32)
                ay = jnp.zeros((L,), jnp.float32)
                for p in range(NR):
                    psl = pl.ds(p * EXT + e * L, L)
                    g = plsc.load_gather(uw2, [ie2[pl.ds(ib + p * EXT + e * L, L)]])
                    ax = ax + g * wxe[psl]
                    ay = ay + g * wye[psl]
                un = uw2[pl.ds(ub + off1 + e * L, L)]
                uxe[sl] = ax - un * Wxe[sl]
                uye[sl] = ay - un * Wye[sl]
                return carry

            lax.fori_loop(0, EXT // L, p1, 0)

            def p2(c, a):
                sl = pl.ds(c * L, L)
                esl = pl.ds(off2 + c * L, L)
                a2 = jnp.zeros((L,), jnp.float32)
                for p in range(NR):
                    g = plsc.load_gather(uxe, [ic[pl.ds(p * CORE + c * L, L)]])
                    a2 = a2 + g * wxe[pl.ds(p * EXT + off2 + c * L, L)]
                uxn = uxe[esl]
                uxx = a2 - uxn * Wxe[esl]
                un = uw2[pl.ds(ub + off3 + c * L, L)]
                fv = uye[esl] - NU * uxx - un * (1.0 - un) * (un + ALPHA)
                d = un - udw2[pl.ds(db + c * L, L)]
                return a + fv * fv * sf + multv[sl] * d * d * sb

            return lax.fori_loop(0, CORE // L, p2, acc)

        issue(0, 0)
        issue(1, 1)

        def pair(kk, acc):
            b0 = 2 * kk
            drain(b0, 0)
            acc = compute(0, acc)
            issue(b0 + 2, 0)
            drain(b0 + 1, 1)
            acc = compute(1, acc)
            issue(b0 + 3, 1)
            return acc

        acc = lax.fori_loop(0, B_PER_SC // 2 - 1, pair,
                            jnp.zeros((L,), jnp.float32))
        drain(B_PER_SC - 2, 0)
        acc = compute(0, acc)
        drain(B_PER_SC - 1, 1)
        acc = compute(1, acc)

        pout[...] = acc
        pltpu.sync_copy(pout, outh.at[pl.ds(pl.multiple_of((sc * NS + tid) * L, 8), L)])

    return k(y2, yd2, xt, invt, pt)


def _reduce_parts(parts):
    def red(x_ref, o_ref):
        o_ref[...] = jnp.sum(x_ref[...]).reshape(1, 1)

    out = pl.pallas_call(
        red, out_shape=jax.ShapeDtypeStruct((1, 1), jnp.float32),
    )(parts)
    return out[0, 0]


@jax.jit
def kernel(y_pred, y_dash, x_f_train, invp_index, p_index):
    y2 = y_pred.reshape(BATCH * N)
    yd2 = y_dash.reshape(BATCH * N)
    xt = x_f_train.T.reshape(2 * N)
    invt = invp_index.reshape(N, 4).T.reshape(4 * N)
    pt = p_index.astype(jnp.int32).T.reshape(NR * N)
    parts = _sc_loss_parts(y2, yd2, xt, invt, pt)
    return _reduce_parts(parts.reshape(NC * NS, L))
